# Initial kernel scaffold; baseline (speedup 1.0000x reference)
import jax
import jax.numpy as jnp
from jax.experimental import pallas as pl


def kernel(x, W_router, w1, w2):
    raise NotImplementedError("write your pallas kernel here")


# speedup vs baseline: 2.3176x; 2.3176x over previous
import functools

import jax
import jax.numpy as jnp
from jax import lax
from jax.experimental import pallas as pl
from jax.experimental.pallas import tpu as pltpu
from jax.experimental.pallas import tpu_sc as plsc

T = 2048
D = 1024
F = 512
E = 64
K = 8

BT = 128
NB = T * K // BT + E
NBT = NB * BT
CT = 256
NC_T = T // CT

SC_NC = 2
SC_NS = 16
NW = SC_NC * SC_NS
TPW = T // NW


def _router_body(x_ref, wr_ref, cw_ref, m_ref, p_ref, cnt_ref, carry):
    c = pl.program_id(0)

    @pl.when(c == 0)
    def _():
        carry[...] = jnp.zeros_like(carry)

    logits = jnp.dot(x_ref[...], wr_ref[...], preferred_element_type=jnp.float32)
    mx = jnp.max(logits, axis=1, keepdims=True)
    ex = jnp.exp(logits - mx)
    probs = ex / jnp.sum(ex, axis=1, keepdims=True)

    iot = lax.broadcasted_iota(jnp.int32, (CT, E), 1)
    wrk = probs
    msel = jnp.zeros((CT, E), dtype=jnp.float32)
    for _ in range(K):
        mxv = jnp.max(wrk, axis=1, keepdims=True)
        cand = jnp.where(wrk == mxv, iot, E)
        amin = jnp.min(cand, axis=1, keepdims=True)
        onehot = (iot == amin).astype(jnp.float32)
        msel = msel + onehot
        wrk = jnp.where(onehot > 0, -1.0, wrk)

    cw_ref[...] = probs * msel
    m_ref[...] = msel

    r_i = lax.broadcasted_iota(jnp.int32, (CT, CT), 0)
    c_i = lax.broadcasted_iota(jnp.int32, (CT, CT), 1)
    ltri = (r_i >= c_i).astype(jnp.float32)
    csum = jnp.dot(ltri, msel, preferred_element_type=jnp.float32)
    base = carry[0:1, :]
    p_ref[...] = base + csum - msel
    new_tot = base + csum[CT - 1 : CT, :]
    carry[0:1, :] = new_tot

    @pl.when(c == NC_T - 1)
    def _():
        cnt_ref[...] = jnp.broadcast_to(new_tot, (8, E))


def _router(x, w_router):
    return pl.pallas_call(
        _router_body,
        grid=(NC_T,),
        in_specs=[
            pl.BlockSpec((CT, D), lambda c: (c, 0)),
            pl.BlockSpec((D, E), lambda c: (0, 0)),
        ],
        out_specs=[
            pl.BlockSpec((CT, E), lambda c: (c, 0)),
            pl.BlockSpec((CT, E), lambda c: (c, 0)),
            pl.BlockSpec((CT, E), lambda c: (c, 0)),
            pl.BlockSpec((8, E), lambda c: (0, 0)),
        ],
        out_shape=[
            jax.ShapeDtypeStruct((T, E), jnp.float32),
            jax.ShapeDtypeStruct((T, E), jnp.float32),
            jax.ShapeDtypeStruct((T, E), jnp.float32),
            jax.ShapeDtypeStruct((8, E), jnp.float32),
        ],
        scratch_shapes=[pltpu.VMEM((8, E), jnp.float32)],
    )(x, w_router)


def _offsets_body(cnt_ref, poff_ref, be_ref):
    cnt = cnt_ref[...]
    pc = jnp.floor((cnt + (BT - 1)) * (1.0 / BT)) * BT
    r_i = lax.broadcasted_iota(jnp.int32, (E, E), 0)
    c_i = lax.broadcasted_iota(jnp.int32, (E, E), 1)
    ustrict = (r_i < c_i).astype(jnp.float32)
    poff = jnp.dot(pc, ustrict, preferred_element_type=jnp.float32)
    poff_ref[...] = poff
    ends = poff[0:1, :] + pc[0:1, :]
    bstart = lax.broadcasted_iota(jnp.int32, (NB, 1), 0).astype(jnp.float32) * BT
    le = (jnp.broadcast_to(ends, (NB, E)) <= bstart).astype(jnp.float32)
    be = jnp.sum(le, axis=1, keepdims=True)
    be_ref[...] = jnp.broadcast_to(be, (NB, 128)).astype(jnp.int32)


def _offsets(cnt):
    return pl.pallas_call(
        _offsets_body,
        grid=(1,),
        in_specs=[pl.BlockSpec((8, E), lambda i: (0, 0))],
        out_specs=[
            pl.BlockSpec((8, E), lambda i: (0, 0)),
            pl.BlockSpec((NB, 128), lambda i: (0, 0)),
        ],
        out_shape=[
            jax.ShapeDtypeStruct((8, E), jnp.float32),
            jax.ShapeDtypeStruct((NB, 128), jnp.int32),
        ],
    )(cnt)


def _pos_body(cw_ref, m_ref, p_ref, poff_ref, pos_ref, wts_ref):
    msel = m_ref[...]
    r_i = lax.broadcasted_iota(jnp.int32, (E, E), 0)
    c_i = lax.broadcasted_iota(jnp.int32, (E, E), 1)
    ustrict = (r_i < c_i).astype(jnp.float32)
    slot = jnp.dot(msel, ustrict, preferred_element_type=jnp.float32)
    pos64 = poff_ref[0:1, :] + p_ref[...]
    cw = cw_ref[...]
    for j in range(K):
        mj = msel * (slot == j).astype(jnp.float32)
        pos_ref[:, j : j + 1] = jnp.sum(pos64 * mj, axis=1, keepdims=True).astype(jnp.int32)
        wts_ref[:, j : j + 1] = jnp.sum(cw * mj, axis=1, keepdims=True)


def _positions(cw, msel, pfx, poff):
    return pl.pallas_call(
        _pos_body,
        grid=(NC_T,),
        in_specs=[
            pl.BlockSpec((CT, E), lambda c: (c, 0)),
            pl.BlockSpec((CT, E), lambda c: (c, 0)),
            pl.BlockSpec((CT, E), lambda c: (c, 0)),
            pl.BlockSpec((8, E), lambda c: (0, 0)),
        ],
        out_specs=[
            pl.BlockSpec((CT, K), lambda c: (c, 0)),
            pl.BlockSpec((CT, K), lambda c: (c, 0)),
        ],
        out_shape=[
            jax.ShapeDtypeStruct((T, K), jnp.int32),
            jax.ShapeDtypeStruct((T, K), jnp.float32),
        ],
    )(cw, msel, pfx, poff)


@functools.lru_cache(maxsize=None)
def _sc_kernels():
    mesh = plsc.VectorSubcoreMesh(core_axis_name="c", subcore_axis_name="s")

    idx_scratch = [pltpu.VMEM((TPW,), jnp.int32) for _ in range(K)]

    @functools.partial(
        pl.kernel,
        mesh=mesh,
        out_type=jax.ShapeDtypeStruct((NBT, D), jnp.float32),
        scratch_types=[
            pltpu.VMEM((TPW, D), jnp.float32),
            *idx_scratch,
            pltpu.SemaphoreType.DMA,
            pltpu.SemaphoreType.DMA,
        ],
    )
    def _dispatch_sc(x_hbm, posf_hbm, xs_hbm, xv, *rest):
        idxs, (isem, sem) = rest[:K], rest[K:]
        wid = lax.axis_index("s") * SC_NC + lax.axis_index("c")
        base = wid * TPW
        pltpu.sync_copy(x_hbm.at[pl.ds(base, TPW)], xv)
        icps = [
            pltpu.async_copy(posf_hbm.at[pl.ds(k * T + base, TPW)], idxs[k], isem)
            for k in range(K)
        ]
        copies = []
        for k in range(K):
            icps[k].wait()
            copies.append(pltpu.async_copy(xv, xs_hbm.at[idxs[k]], sem))
        for cp in copies:
            cp.wait()

    HW = TPW // 2

    @functools.partial(
        pl.kernel,
        mesh=mesh,
        out_type=jax.ShapeDtypeStruct((K, T, D), jnp.float32),
        scratch_types=[
            pltpu.VMEM((HW, D), jnp.float32),
            pltpu.VMEM((HW, D), jnp.float32),
            *idx_scratch,
            pltpu.SemaphoreType.DMA,
            pltpu.SemaphoreType.DMA,
            pltpu.SemaphoreType.DMA,
        ],
    )
    def _gather_sc(y_hbm, posf_hbm, yg_hbm, gb0, gb1, *rest):
        idxs, (isem, gsem, wsem) = rest[:K], rest[K:]
        wid = lax.axis_index("s") * SC_NC + lax.axis_index("c")
        base = wid * TPW
        icps = [
            pltpu.async_copy(posf_hbm.at[pl.ds(k * T + base, TPW)], idxs[k], isem)
            for k in range(K)
        ]
        gbufs = (gb0, gb1)
        wcps = [None, None]
        step = 0
        for k in range(K):
            icps[k].wait()
            for h in range(2):
                b = step % 2
                if wcps[b] is not None:
                    wcps[b].wait()
                pltpu.async_copy(
                    y_hbm.at[idxs[k].at[pl.ds(h * HW, HW)]], gbufs[b], gsem
                ).wait()
                wcps[b] = pltpu.async_copy(
                    gbufs[b], yg_hbm.at[k, pl.ds(base + h * HW, HW)], wsem
                )
                step += 1
        for w in wcps:
            w.wait()

    return _dispatch_sc, _gather_sc


def _ffn_body(be_ref, xs_ref, w1_ref, w2_ref, y_ref):
    @pl.when(be_ref[pl.program_id(0)] < E)
    def _():
        h = jnp.maximum(
            jnp.dot(xs_ref[...], w1_ref[0], preferred_element_type=jnp.float32), 0.0
        )
        y_ref[...] = jnp.dot(h, w2_ref[0], preferred_element_type=jnp.float32)


def _ffn(be, xs, w1, w2):
    grid_spec = pltpu.PrefetchScalarGridSpec(
        num_scalar_prefetch=1,
        grid=(NB,),
        in_specs=[
            pl.BlockSpec((BT, D), lambda b, be: (b, 0)),
            pl.BlockSpec((1, D, F), lambda b, be: (jnp.minimum(be[b], E - 1), 0, 0)),
            pl.BlockSpec((1, F, D), lambda b, be: (jnp.minimum(be[b], E - 1), 0, 0)),
        ],
        out_specs=pl.BlockSpec((BT, D), lambda b, be: (b, 0)),
    )
    return pl.pallas_call(
        _ffn_body,
        grid_spec=grid_spec,
        out_shape=jax.ShapeDtypeStruct((NBT, D), jnp.float32),
    )(be, xs, w1, w2)


def _combine_body(yg_ref, w_ref, o_ref):
    acc = w_ref[:, 0:1] * yg_ref[0]
    for k in range(1, K):
        acc = acc + w_ref[:, k : k + 1] * yg_ref[k]
    o_ref[...] = acc


def _combine(yg, wts):
    return pl.pallas_call(
        _combine_body,
        grid=(NC_T,),
        in_specs=[
            pl.BlockSpec((K, CT, D), lambda c: (0, c, 0)),
            pl.BlockSpec((CT, K), lambda c: (c, 0)),
        ],
        out_specs=pl.BlockSpec((CT, D), lambda c: (c, 0)),
        out_shape=jax.ShapeDtypeStruct((T, D), jnp.float32),
    )(yg, wts)


def kernel(x, W_router, w1, w2):
    cw, msel, pfx, cnt = _router(x, W_router)
    poff, be2d = _offsets(cnt)
    pos, wts = _positions(cw, msel, pfx, poff)
    posf = jnp.ravel(pos.T)
    be = be2d[:, 0]
    dispatch_sc, gather_sc = _sc_kernels()
    xs = dispatch_sc(x, posf)
    y = _ffn(be, xs, w1, w2)
    yg = gather_sc(y, posf)
    return _combine(yg, wts)
